# baseline (speedup 1.0000x reference)
import jax
import jax.numpy as jnp
from jax.experimental import pallas as pl
from jax.experimental.pallas import tpu as pltpu

_S, _D = 2048, 768
_E, _F = 8, 3072

_BM = 256
_FC = 1536
_NT = _S // _BM
_NC = _F // _FC


def _router_kernel(x_ref, wr_ref, w_ref):
    x = x_ref[...]
    wr = wr_ref[...]
    logits = jax.lax.dot_general(
        x, wr, (((1,), (1,)), ((), ())),
        preferred_element_type=jnp.float32)
    m = jnp.max(logits, axis=1, keepdims=True)
    ex = jnp.exp(logits - m)
    p = ex / jnp.sum(ex, axis=1, keepdims=True)

    eidx = jax.lax.broadcasted_iota(jnp.int32, (_S, _E), 1)
    p1 = jnp.max(p, axis=1, keepdims=True)
    i1 = jnp.min(jnp.where(p == p1, eidx, _E), axis=1, keepdims=True)
    pm = jnp.where(eidx == i1, -1.0, p)
    p2 = jnp.max(pm, axis=1, keepdims=True)
    i2 = jnp.min(jnp.where(pm == p2, eidx, _E), axis=1, keepdims=True)
    s = p1 + p2
    w = jnp.where(eidx == i1, p1 / s, 0.0) + jnp.where(eidx == i2, p2 / s, 0.0)
    w_ref[...] = w


def _ffn_kernel(w_ref, x_ref, w1_ref, b1_ref, w2_ref, b2_ref, out_ref):
    e = pl.program_id(0)
    c = pl.program_id(1)
    t = pl.program_id(2)

    @pl.when((e == 0) & (c == 0) & (t == 0))
    def _init():
        out_ref[...] = jnp.zeros_like(out_ref)

    x = x_ref[pl.ds(t * _BM, _BM), :]
    h = jax.lax.dot_general(
        x, w1_ref[0], (((1,), (0,)), ((), ())),
        preferred_element_type=jnp.float32) + b1_ref[0]
    h = jax.nn.gelu(h)
    y = jax.lax.dot_general(
        h, w2_ref[0], (((1,), (0,)), ((), ())),
        preferred_element_type=jnp.float32)

    wcol = jnp.sum(
        w_ref[pl.ds(t * _BM, _BM), :]
        * (jax.lax.broadcasted_iota(jnp.int32, (_BM, _E), 1) == e),
        axis=1, keepdims=True)
    contrib = wcol * y
    @pl.when(c == 0)
    def _bias():
        out_ref[pl.ds(t * _BM, _BM), :] += wcol * b2_ref[0]
    out_ref[pl.ds(t * _BM, _BM), :] += contrib


def kernel(hidden_states, Wr, W1, b1, W2, b2):
    bsz, seq_len, dim = hidden_states.shape
    x = hidden_states.reshape(-1, dim)

    w = pl.pallas_call(
        _router_kernel,
        out_shape=jax.ShapeDtypeStruct((_S, _E), jnp.float32),
    )(x, Wr)

    out = pl.pallas_call(
        _ffn_kernel,
        grid=(_E, _NC, _NT),
        in_specs=[
            pl.BlockSpec((_S, _E), lambda e, c, t: (0, 0)),
            pl.BlockSpec((_S, _D), lambda e, c, t: (0, 0)),
            pl.BlockSpec((1, _D, _FC), lambda e, c, t: (e, 0, c)),
            pl.BlockSpec((1, 1, _FC), lambda e, c, t: (e, 0, c)),
            pl.BlockSpec((1, _FC, _D), lambda e, c, t: (e, c, 0)),
            pl.BlockSpec((1, 1, _D), lambda e, c, t: (e, 0, 0)),
        ],
        out_specs=pl.BlockSpec((_S, _D), lambda e, c, t: (0, 0)),
        out_shape=jax.ShapeDtypeStruct((_S, _D), jnp.float32),
        compiler_params=pltpu.CompilerParams(
            dimension_semantics=("arbitrary", "arbitrary", "arbitrary"),
        ),
    )(w, x, W1, b1.reshape(_E, 1, _F), W2, b2.reshape(_E, 1, _D))

    return out.reshape(bsz, seq_len, dim)


# speedup vs baseline: 1.5218x; 1.5218x over previous
import jax
import jax.numpy as jnp
from jax import lax
from jax.experimental import pallas as pl
from jax.experimental.pallas import tpu as pltpu
from jax.experimental.pallas import tpu_sc as plsc

_S, _D = 2048, 768
_E, _F = 8, 3072

_BM = 128
_NG = (_S * 2) // _BM + _E
_NGP = 48
_NSP = _NG * _BM
_NW = 32
_BPW = _S // _NW


def _router_plan_kernel(x_ref, wr_ref, pos1_ref, pos2_ref, w1b_ref, w2b_ref,
                        be_ref):
    x = x_ref[...]
    wr = wr_ref[...]
    logits = lax.dot_general(
        x, wr, (((1,), (1,)), ((), ())),
        preferred_element_type=jnp.float32)
    m = jnp.max(logits, axis=1, keepdims=True)
    ex = jnp.exp(logits - m)
    p = ex / jnp.sum(ex, axis=1, keepdims=True)

    eidx = lax.broadcasted_iota(jnp.int32, (_S, _E), 1)
    p1 = jnp.max(p, axis=1, keepdims=True)
    i1 = jnp.min(jnp.where(p == p1, eidx, _E), axis=1, keepdims=True)
    pm = jnp.where(eidx == i1, -1.0, p)
    p2 = jnp.max(pm, axis=1, keepdims=True)
    i2 = jnp.min(jnp.where(pm == p2, eidx, _E), axis=1, keepdims=True)
    s = p1 + p2
    ones16 = jnp.ones((1, 16), jnp.float32)
    w1b_ref[...] = (p1 / s) * ones16
    w2b_ref[...] = (p2 / s) * ones16

    mask = (eidx == i1).astype(jnp.float32) + (eidx == i2).astype(jnp.float32)
    counts = jnp.sum(mask, axis=0, keepdims=True)
    nbk = jnp.ceil(counts / _BM)
    lt8 = (lax.broadcasted_iota(jnp.int32, (_E, _E), 0)
           < lax.broadcasted_iota(jnp.int32, (_E, _E), 1)).astype(jnp.float32)
    boff = lax.dot_general(nbk, lt8, (((1,), (0,)), ((), ())),
                           preferred_element_type=jnp.float32)

    lts = (lax.broadcasted_iota(jnp.int32, (_S, _S), 1)
           < lax.broadcasted_iota(jnp.int32, (_S, _S), 0)).astype(jnp.float32)
    rank = lax.dot_general(lts, mask, (((1,), (0,)), ((), ())),
                           preferred_element_type=jnp.float32)
    posmat = boff * _BM + rank
    pos1 = jnp.sum(jnp.where(eidx == i1, posmat, 0.0), axis=1, keepdims=True)
    pos2 = jnp.sum(jnp.where(eidx == i2, posmat, 0.0), axis=1, keepdims=True)
    pos1_ref[...] = pos1.astype(jnp.int32)
    pos2_ref[...] = pos2.astype(jnp.int32)

    g_i = lax.broadcasted_iota(jnp.int32, (_NGP, _E), 0).astype(jnp.float32)
    cond = (boff <= g_i) & (g_i < boff + nbk)
    e_col = lax.broadcasted_iota(jnp.int32, (_NGP, _E), 1).astype(jnp.float32)
    be_col = jnp.sum(jnp.where(cond, e_col, 0.0), axis=1, keepdims=True)
    nb = jnp.sum(nbk)
    row_i = lax.broadcasted_iota(jnp.int32, (_NGP, 1), 0).astype(jnp.float32)
    belast = jnp.sum(jnp.where(row_i == nb - 1.0, be_col, 0.0))
    be_out = jnp.where(row_i < nb, be_col, belast)
    be_out = jnp.where(row_i == _NG, nb, be_out)
    be_ref[...] = be_out.astype(jnp.int32)


def _ffn_kernel(be_ref, nb_ref, xd_ref, w1_ref, b1_ref, w2_ref, b2_ref,
                yd_ref):
    g = pl.program_id(0)

    @pl.when(g < nb_ref[0])
    def _():
        xb = xd_ref[...]
        h = lax.dot_general(xb, w1_ref[0], (((1,), (0,)), ((), ())),
                            preferred_element_type=jnp.float32) + b1_ref[0]
        h = jax.nn.gelu(h)
        yd_ref[...] = lax.dot_general(
            h, w2_ref[0], (((1,), (0,)), ((), ())),
            preferred_element_type=jnp.float32) + b2_ref[0]


def _sc_mesh():
    return plsc.VectorSubcoreMesh(core_axis_name="c", subcore_axis_name="s",
                                  num_cores=2, num_subcores=16)


def _dispatch_body(x_hbm, p1_hbm, p2_hbm, xd_hbm, i1_v, i2_v, x_v, sem):
    wid = lax.axis_index("s") * 2 + lax.axis_index("c")
    base = wid * _BPW
    pltpu.sync_copy(p1_hbm.at[wid], i1_v)
    pltpu.sync_copy(p2_hbm.at[wid], i2_v)
    pltpu.sync_copy(x_hbm.at[pl.ds(base, _BPW)], x_v)
    pltpu.async_copy(x_v, xd_hbm.at[i1_v], sem).wait()
    pltpu.async_copy(x_v, xd_hbm.at[i2_v], sem).wait()


def _combine_body(yd_hbm, p1_hbm, p2_hbm, w1_hbm, w2_hbm, out_hbm,
                  i1_v, i2_v, y1_v, y2_v, w1_v, w2_v, sem):
    wid = lax.axis_index("s") * 2 + lax.axis_index("c")
    base = wid * _BPW
    pltpu.sync_copy(p1_hbm.at[wid], i1_v)
    pltpu.sync_copy(p2_hbm.at[wid], i2_v)
    pltpu.sync_copy(w1_hbm.at[pl.ds(base, _BPW)], w1_v)
    pltpu.sync_copy(w2_hbm.at[pl.ds(base, _BPW)], w2_v)
    pltpu.async_copy(yd_hbm.at[i1_v], y1_v, sem).wait()
    pltpu.async_copy(yd_hbm.at[i2_v], y2_v, sem).wait()

    @pl.loop(0, _BPW)
    def _row(i):
        w1v = w1_v.at[pl.ds(i, 1), :][...]
        w2v = w2_v.at[pl.ds(i, 1), :][...]

        @pl.loop(0, _D, step=16)
        def _chunk(c):
            slc = (pl.ds(i, 1), pl.ds(c, 16))
            y1_v.at[slc][...] = (w1v * y1_v.at[slc][...]
                                 + w2v * y2_v.at[slc][...])

    pltpu.sync_copy(y1_v, out_hbm.at[pl.ds(base, _BPW)])


def kernel(hidden_states, Wr, W1, b1, W2, b2):
    bsz, seq_len, dim = hidden_states.shape
    x = hidden_states.reshape(_S, _D)

    pos1, pos2, w1b, w2b, beplan = pl.pallas_call(
        _router_plan_kernel,
        out_shape=[
            jax.ShapeDtypeStruct((_S, 1), jnp.int32),
            jax.ShapeDtypeStruct((_S, 1), jnp.int32),
            jax.ShapeDtypeStruct((_S, 16), jnp.float32),
            jax.ShapeDtypeStruct((_S, 16), jnp.float32),
            jax.ShapeDtypeStruct((_NGP, 1), jnp.int32),
        ],
    )(x, Wr)

    beflat = beplan.reshape(_NGP)
    be = beflat[:_NG]
    nbv = beflat[_NG:_NG + 1]
    p1w = pos1.reshape(_NW, _BPW)
    p2w = pos2.reshape(_NW, _BPW)

    xd = pl.kernel(
        _dispatch_body,
        out_type=jax.ShapeDtypeStruct((_NSP, _D), jnp.float32),
        mesh=_sc_mesh(),
        scratch_types=[
            pltpu.VMEM((_BPW,), jnp.int32),
            pltpu.VMEM((_BPW,), jnp.int32),
            pltpu.VMEM((_BPW, _D), jnp.float32),
            pltpu.SemaphoreType.DMA,
        ],
    )(x, p1w, p2w)

    yd = pl.pallas_call(
        _ffn_kernel,
        grid_spec=pltpu.PrefetchScalarGridSpec(
            num_scalar_prefetch=2,
            grid=(_NG,),
            in_specs=[
                pl.BlockSpec((_BM, _D), lambda g, be, nb: (g, 0)),
                pl.BlockSpec((1, _D, _F), lambda g, be, nb: (be[g], 0, 0)),
                pl.BlockSpec((1, 1, _F), lambda g, be, nb: (be[g], 0, 0)),
                pl.BlockSpec((1, _F, _D), lambda g, be, nb: (be[g], 0, 0)),
                pl.BlockSpec((1, 1, _D), lambda g, be, nb: (be[g], 0, 0)),
            ],
            out_specs=pl.BlockSpec((_BM, _D), lambda g, be, nb: (g, 0)),
        ),
        out_shape=jax.ShapeDtypeStruct((_NSP, _D), jnp.float32),
        compiler_params=pltpu.CompilerParams(
            dimension_semantics=("arbitrary",),
        ),
    )(be, nbv, xd, W1, b1.reshape(_E, 1, _F), W2, b2.reshape(_E, 1, _D))

    out = pl.kernel(
        _combine_body,
        out_type=jax.ShapeDtypeStruct((_S, _D), jnp.float32),
        mesh=_sc_mesh(),
        scratch_types=[
            pltpu.VMEM((_BPW,), jnp.int32),
            pltpu.VMEM((_BPW,), jnp.int32),
            pltpu.VMEM((_BPW, _D), jnp.float32),
            pltpu.VMEM((_BPW, _D), jnp.float32),
            pltpu.VMEM((_BPW, 16), jnp.float32),
            pltpu.VMEM((_BPW, 16), jnp.float32),
            pltpu.SemaphoreType.DMA,
        ],
    )(yd, p1w, p2w, w1b, w2b)

    return out.reshape(bsz, seq_len, dim)


# speedup vs baseline: 1.5228x; 1.0006x over previous
import jax
import jax.numpy as jnp
from jax import lax
from jax.experimental import pallas as pl
from jax.experimental.pallas import tpu as pltpu
from jax.experimental.pallas import tpu_sc as plsc

_S, _D = 2048, 768
_E, _F = 8, 3072

_BM = 128
_NG = (_S * 2) // _BM + _E
_NGP = 48
_NSP = _NG * _BM
_NW = 32
_BPW = _S // _NW


def _router_plan_kernel(x_ref, wr_ref, pos1_ref, pos2_ref, w1b_ref, w2b_ref,
                        be_ref):
    x = x_ref[...]
    wr = wr_ref[...]
    logits = lax.dot_general(
        x, wr, (((1,), (1,)), ((), ())),
        preferred_element_type=jnp.float32)
    m = jnp.max(logits, axis=1, keepdims=True)
    ex = jnp.exp(logits - m)
    p = ex / jnp.sum(ex, axis=1, keepdims=True)

    eidx = lax.broadcasted_iota(jnp.int32, (_S, _E), 1)
    p1 = jnp.max(p, axis=1, keepdims=True)
    i1 = jnp.min(jnp.where(p == p1, eidx, _E), axis=1, keepdims=True)
    pm = jnp.where(eidx == i1, -1.0, p)
    p2 = jnp.max(pm, axis=1, keepdims=True)
    i2 = jnp.min(jnp.where(pm == p2, eidx, _E), axis=1, keepdims=True)
    s = p1 + p2
    ones16 = jnp.ones((1, 16), jnp.float32)
    w1b_ref[...] = (p1 / s) * ones16
    w2b_ref[...] = (p2 / s) * ones16

    mask = (eidx == i1).astype(jnp.float32) + (eidx == i2).astype(jnp.float32)
    counts = jnp.sum(mask, axis=0, keepdims=True)
    nbk = jnp.ceil(counts / _BM)
    lt8 = (lax.broadcasted_iota(jnp.int32, (_E, _E), 0)
           < lax.broadcasted_iota(jnp.int32, (_E, _E), 1)).astype(jnp.float32)
    boff = lax.dot_general(nbk, lt8, (((1,), (0,)), ((), ())),
                           preferred_element_type=jnp.float32)

    lts = (lax.broadcasted_iota(jnp.int32, (_S, _S), 1)
           < lax.broadcasted_iota(jnp.int32, (_S, _S), 0)).astype(jnp.float32)
    rank = lax.dot_general(lts, mask, (((1,), (0,)), ((), ())),
                           preferred_element_type=jnp.float32)
    posmat = boff * _BM + rank
    pos1 = jnp.sum(jnp.where(eidx == i1, posmat, 0.0), axis=1, keepdims=True)
    pos2 = jnp.sum(jnp.where(eidx == i2, posmat, 0.0), axis=1, keepdims=True)
    pos1_ref[...] = pos1.astype(jnp.int32)
    pos2_ref[...] = pos2.astype(jnp.int32)

    g_i = lax.broadcasted_iota(jnp.int32, (_NGP, _E), 0).astype(jnp.float32)
    cond = (boff <= g_i) & (g_i < boff + nbk)
    e_col = lax.broadcasted_iota(jnp.int32, (_NGP, _E), 1).astype(jnp.float32)
    be_col = jnp.sum(jnp.where(cond, e_col, 0.0), axis=1, keepdims=True)
    nb = jnp.sum(nbk)
    row_i = lax.broadcasted_iota(jnp.int32, (_NGP, 1), 0).astype(jnp.float32)
    belast = jnp.sum(jnp.where(row_i == nb - 1.0, be_col, 0.0))
    be_out = jnp.where(row_i < nb, be_col, belast)
    be_out = jnp.where(row_i == _NG, nb, be_out)
    be_ref[...] = be_out.astype(jnp.int32)


def _ffn_kernel(be_ref, nb_ref, xd_ref, w1_ref, b1_ref, w2_ref, b2_ref,
                yd_ref):
    g = pl.program_id(0)

    @pl.when(g < nb_ref[0])
    def _():
        xb = xd_ref[...]
        h = lax.dot_general(xb, w1_ref[0], (((1,), (0,)), ((), ())),
                            preferred_element_type=jnp.float32) + b1_ref[0]
        h = jax.nn.gelu(h)
        yd_ref[...] = lax.dot_general(
            h, w2_ref[0], (((1,), (0,)), ((), ())),
            preferred_element_type=jnp.float32) + b2_ref[0]


def _sc_mesh():
    return plsc.VectorSubcoreMesh(core_axis_name="c", subcore_axis_name="s",
                                  num_cores=2, num_subcores=16)


def _dispatch_body(x_hbm, p1_hbm, p2_hbm, xd_hbm, i1_v, i2_v, x_v, sem):
    wid = lax.axis_index("s") * 2 + lax.axis_index("c")
    base = wid * _BPW
    pltpu.sync_copy(p1_hbm.at[wid], i1_v)
    pltpu.sync_copy(p2_hbm.at[wid], i2_v)
    pltpu.sync_copy(x_hbm.at[pl.ds(base, _BPW)], x_v)
    pltpu.async_copy(x_v, xd_hbm.at[i1_v], sem).wait()
    pltpu.async_copy(x_v, xd_hbm.at[i2_v], sem).wait()


def _combine_body(yd_hbm, p1_hbm, p2_hbm, w1_hbm, w2_hbm, out_hbm,
                  i1_v, i2_v, y1_v, y2_v, w1_v, w2_v, sem):
    wid = lax.axis_index("s") * 2 + lax.axis_index("c")
    base = wid * _BPW
    pltpu.sync_copy(p1_hbm.at[wid], i1_v)
    pltpu.sync_copy(p2_hbm.at[wid], i2_v)
    pltpu.sync_copy(w1_hbm.at[pl.ds(base, _BPW)], w1_v)
    pltpu.sync_copy(w2_hbm.at[pl.ds(base, _BPW)], w2_v)
    pltpu.async_copy(yd_hbm.at[i1_v], y1_v, sem).wait()
    pltpu.async_copy(yd_hbm.at[i2_v], y2_v, sem).wait()

    @pl.loop(0, _BPW)
    def _row(i):
        w1v = w1_v.at[pl.ds(i, 1), :][...]
        w2v = w2_v.at[pl.ds(i, 1), :][...]

        @pl.loop(0, _D, step=16)
        def _chunk(c):
            slc = (pl.ds(i, 1), pl.ds(c, 16))
            y1_v.at[slc][...] = (w1v * y1_v.at[slc][...]
                                 + w2v * y2_v.at[slc][...])

    pltpu.sync_copy(y1_v, out_hbm.at[pl.ds(base, _BPW)])


def kernel(hidden_states, Wr, W1, b1, W2, b2):
    bsz, seq_len, dim = hidden_states.shape
    x = hidden_states.reshape(_S, _D)

    pos1, pos2, w1b, w2b, beplan = pl.pallas_call(
        _router_plan_kernel,
        out_shape=[
            jax.ShapeDtypeStruct((_S, 1), jnp.int32),
            jax.ShapeDtypeStruct((_S, 1), jnp.int32),
            jax.ShapeDtypeStruct((_S, 16), jnp.float32),
            jax.ShapeDtypeStruct((_S, 16), jnp.float32),
            jax.ShapeDtypeStruct((_NGP, 1), jnp.int32),
        ],
    )(x, Wr)

    beflat = beplan.reshape(_NGP)
    be = beflat[:_NG]
    nbv = beflat[_NG:_NG + 1]
    p1w = pos1.reshape(_NW, _BPW)
    p2w = pos2.reshape(_NW, _BPW)

    xd = pl.kernel(
        _dispatch_body,
        out_type=jax.ShapeDtypeStruct((_NSP, _D), jnp.float32),
        mesh=_sc_mesh(),
        scratch_types=[
            pltpu.VMEM((_BPW,), jnp.int32),
            pltpu.VMEM((_BPW,), jnp.int32),
            pltpu.VMEM((_BPW, _D), jnp.float32),
            pltpu.SemaphoreType.DMA,
        ],
    )(x, p1w, p2w)

    yd = pl.pallas_call(
        _ffn_kernel,
        grid_spec=pltpu.PrefetchScalarGridSpec(
            num_scalar_prefetch=2,
            grid=(_NG,),
            in_specs=[
                pl.BlockSpec((_BM, _D), lambda g, be, nb: (g, 0)),
                pl.BlockSpec((1, _D, _F), lambda g, be, nb: (be[g], 0, 0)),
                pl.BlockSpec((1, 1, _F), lambda g, be, nb: (be[g], 0, 0)),
                pl.BlockSpec((1, _F, _D), lambda g, be, nb: (be[g], 0, 0)),
                pl.BlockSpec((1, 1, _D), lambda g, be, nb: (be[g], 0, 0)),
            ],
            out_specs=pl.BlockSpec((_BM, _D), lambda g, be, nb: (g, 0)),
        ),
        out_shape=jax.ShapeDtypeStruct((_NSP, _D), jnp.float32),
        compiler_params=pltpu.CompilerParams(
            dimension_semantics=("parallel",),
        ),
    )(be, nbv, xd, W1, b1.reshape(_E, 1, _F), W2, b2.reshape(_E, 1, _D))

    out = pl.kernel(
        _combine_body,
        out_type=jax.ShapeDtypeStruct((_S, _D), jnp.float32),
        mesh=_sc_mesh(),
        scratch_types=[
            pltpu.VMEM((_BPW,), jnp.int32),
            pltpu.VMEM((_BPW,), jnp.int32),
            pltpu.VMEM((_BPW, _D), jnp.float32),
            pltpu.VMEM((_BPW, _D), jnp.float32),
            pltpu.VMEM((_BPW, 16), jnp.float32),
            pltpu.VMEM((_BPW, 16), jnp.float32),
            pltpu.SemaphoreType.DMA,
        ],
    )(yd, p1w, p2w, w1b, w2b)

    return out.reshape(bsz, seq_len, dim)
